# baseline (speedup 1.0000x reference)
import functools

import jax
import jax.numpy as jnp
from jax.experimental import pallas as pl
from jax.experimental.pallas import tpu as pltpu

EPS = 1.1920929e-07

BL = 512
BM = 512
BN = 512


def _linear_kernel(x_ref, w_ref, b_ref, glh_ref, sq_ref, sqe_ref):
    gl = jnp.maximum(
        jnp.dot(x_ref[...], w_ref[...], preferred_element_type=jnp.float32)
        + b_ref[...],
        0.0,
    )
    glh_ref[...] = gl.astype(jnp.bfloat16)
    sq = jnp.sum(gl * gl, axis=1, keepdims=True)
    sq_ref[...] = sq
    sqe_ref[...] = sq + EPS


def _rowfold(p):
    acc = p[:, 0:128]
    for c in range(128, BN, 128):
        acc = acc + p[:, c:c + 128]
    return acc


def _mega_kernel(glh_ref, sqe_ref, sqt_ref, eye_ref, x_ref, w_ref, b_ref,
                 prob_ref, out_ref,
                 pb_s, rs_s, dinv_s, uh_s, tx1_s, u2h_s, *, NI, NJ):
    t = pl.program_id(0)

    @pl.when(t < NI * NJ)
    def _():
        i = t // NJ
        j = t - i * NJ
        glr = glh_ref[pl.ds(i * BM, BM), :]
        glc = glh_ref[pl.ds(j * BN, BN), :]
        sq_r = sqe_ref[pl.ds(i * BM, BM), :]
        sq_c = sqt_ref[j]
        g2 = jax.lax.dot_general(
            glr * jnp.bfloat16(-2.0), glc, (((1,), (1,)), ((), ())),
            preferred_element_type=jnp.float32,
        )
        diff = jnp.maximum((g2 + sq_c) + sq_r, EPS)
        prob = jnp.exp2((diff * jnp.float32(-1.4426950408889634))
                        * jax.lax.rsqrt(diff))
        def _emit(p):
            prob_ref[...] = p
            pb_s[j, pl.ds(i * BM, BM), :] = p.astype(jnp.bfloat16)
            ps = _rowfold(p)

            @pl.when(j == 0)
            def _():
                rs_s[pl.ds(i * BM, BM), :] = ps

            @pl.when(j != 0)
            def _():
                rs_s[pl.ds(i * BM, BM), :] += ps

        @pl.when(i == j)
        def _():
            _emit(jnp.maximum(prob, eye_ref[...]))

        @pl.when(i != j)
        def _():
            _emit(prob)

        @pl.when(j == NJ - 1)
        def _():
            deg = jnp.sum(rs_s[pl.ds(i * BM, BM), :], axis=1,
                          keepdims=True) - 1.0
            dinv = jnp.where(deg > 0.0, jax.lax.rsqrt(deg), 0.0)
            dinv_s[pl.ds(i * BM, BM), :] = dinv
            uh_s[pl.ds(i * BM, BM), :] = (
                dinv * x_ref[pl.ds(i * BM, BM), :]).astype(jnp.bfloat16)

    @pl.when(jnp.logical_and(t >= NI * NJ, t < NI * NJ + NI))
    def _():
        i = t - NI * NJ
        z = jnp.zeros((BM, 128), jnp.float32)
        for jj in range(NJ):
            z += jax.lax.dot_general(
                pb_s[jj, pl.ds(i * BM, BM), :],
                uh_s[pl.ds(jj * BN, BN), :],
                (((1,), (0,)), ((), ())),
                preferred_element_type=jnp.float32,
            )
        u_i = uh_s[pl.ds(i * BM, BM), :].astype(jnp.float32)
        dinv_i = dinv_s[pl.ds(i * BM, BM), :]
        tx1 = -(dinv_i * (z - u_i))
        tx1_s[pl.ds(i * BM, BM), :] = tx1
        u2h_s[pl.ds(i * BM, BM), :] = (dinv_i * tx1).astype(jnp.bfloat16)

    @pl.when(t >= NI * NJ + NI)
    def _():
        i = t - (NI * NJ + NI)
        z = jnp.zeros((BM, 128), jnp.float32)
        for jj in range(NJ):
            z += jax.lax.dot_general(
                pb_s[jj, pl.ds(i * BM, BM), :],
                u2h_s[pl.ds(jj * BN, BN), :],
                (((1,), (0,)), ((), ())),
                preferred_element_type=jnp.float32,
            )
        u2_i = u2h_s[pl.ds(i * BM, BM), :].astype(jnp.float32)
        dinv_i = dinv_s[pl.ds(i * BM, BM), :]
        lt = -(dinv_i * (z - u2_i))
        xi = x_ref[pl.ds(i * BM, BM), :]
        tx1i = tx1_s[pl.ds(i * BM, BM), :]
        tx2 = 2.0 * lt - xi
        o = jnp.dot(xi, w_ref[0], preferred_element_type=jnp.float32)
        o += jnp.dot(tx1i, w_ref[1], preferred_element_type=jnp.float32)
        o += jnp.dot(tx2, w_ref[2], preferred_element_type=jnp.float32)
        out_ref[...] = jnp.maximum(o + b_ref[...], 0.0)


def kernel(input, adj, lin_w, lin_b, cheb_w, cheb_b):
    x = input
    n, d = x.shape

    glh, sq, sqe = pl.pallas_call(
        _linear_kernel,
        grid=(n // BL,),
        in_specs=[
            pl.BlockSpec((BL, d), lambda i: (i, 0)),
            pl.BlockSpec((d, d), lambda i: (0, 0)),
            pl.BlockSpec((1, d), lambda i: (0, 0)),
        ],
        out_specs=[
            pl.BlockSpec((BL, d), lambda i: (i, 0)),
            pl.BlockSpec((BL, 1), lambda i: (i, 0)),
            pl.BlockSpec((BL, 1), lambda i: (i, 0)),
        ],
        out_shape=[
            jax.ShapeDtypeStruct((n, d), jnp.bfloat16),
            jax.ShapeDtypeStruct((n, 1), jnp.float32),
            jax.ShapeDtypeStruct((n, 1), jnp.float32),
        ],
        compiler_params=pltpu.CompilerParams(
            dimension_semantics=("parallel",)),
    )(x, lin_w, lin_b.reshape(1, d))

    NI, NJ = n // BM, n // BN
    sqt = sq.reshape(NJ, 1, BN)
    eye = jnp.eye(BM, dtype=jnp.float32)
    nt = NI * NJ + 2 * NI

    full = lambda t: (0, 0)

    prob, out = pl.pallas_call(
        functools.partial(_mega_kernel, NI=NI, NJ=NJ),
        grid=(nt,),
        in_specs=[
            pl.BlockSpec((n, d), full),
            pl.BlockSpec((n, 1), full),
            pl.BlockSpec((NJ, 1, BN), lambda t: (0, 0, 0)),

            pl.BlockSpec((BM, BN), full),
            pl.BlockSpec((n, d), full),
            pl.BlockSpec((3, d, d), lambda t: (0, 0, 0)),
            pl.BlockSpec((1, d), full),
        ],
        out_specs=[
            pl.BlockSpec(
                (BM, BN),
                lambda t, NI=NI, NJ=NJ: (
                    jnp.where(t < NI * NJ, t // NJ, NI - 1),
                    jnp.where(t < NI * NJ, t % NJ, NJ - 1)),
            ),
            pl.BlockSpec(
                (BM, d),
                lambda t, NI=NI, NJ=NJ: (
                    jnp.where(t >= NI * NJ + NI, t - (NI * NJ + NI), 0), 0),
            ),
        ],
        out_shape=[
            jax.ShapeDtypeStruct((n, n), jnp.float32),
            jax.ShapeDtypeStruct((n, d), jnp.float32),
        ],
        scratch_shapes=[
            pltpu.VMEM((NJ, n, BN), jnp.bfloat16),
            pltpu.VMEM((n, 128), jnp.float32),
            pltpu.VMEM((n, 1), jnp.float32),
            pltpu.VMEM((n, d), jnp.bfloat16),
            pltpu.VMEM((n, d), jnp.float32),
            pltpu.VMEM((n, d), jnp.bfloat16),
        ],
        compiler_params=pltpu.CompilerParams(
            dimension_semantics=("arbitrary",)),
    )(glh, sqe, sqt, eye, x, cheb_w, cheb_b.reshape(1, d))

    return out, prob


# speedup vs baseline: 1.0396x; 1.0396x over previous
import functools

import jax
import jax.numpy as jnp
from jax.experimental import pallas as pl
from jax.experimental.pallas import tpu as pltpu

EPS = 1.1920929e-07

BL = 512
BM = 512
BN = 512


def _linear_kernel(x_ref, w_ref, b_ref, glh_ref, sq_ref, sqe_ref):
    gl = jnp.maximum(
        jnp.dot(x_ref[...], w_ref[...], preferred_element_type=jnp.float32)
        + b_ref[...],
        0.0,
    )
    glh_ref[...] = gl.astype(jnp.bfloat16)
    sq = jnp.sum(gl * gl, axis=1, keepdims=True)
    sq_ref[...] = sq
    sqe_ref[...] = sq + EPS


def _rowfold(p):
    acc = p[:, 0:128]
    for c in range(128, BN, 128):
        acc = acc + p[:, c:c + 128]
    return acc


def _mega_kernel(glh_ref, sqe_ref, sqt_ref, eye_ref, x_ref, w_ref, b_ref,
                 prob_ref, out_ref,
                 pb_s, rs_s, dinv_s, uh_s, tx1_s, u2h_s, *, NI, NJ):
    t = pl.program_id(0)

    @pl.when(t < NI * NJ)
    def _():
        i = t // NJ
        j = t - i * NJ
        glr = glh_ref[pl.ds(i * BM, BM), :]
        glc = glh_ref[pl.ds(j * BN, BN), :]
        sq_r = sqe_ref[pl.ds(i * BM, BM), :]
        sq_c = sqt_ref[j]
        g2 = jax.lax.dot_general(
            glr * jnp.bfloat16(-2.0), glc, (((1,), (1,)), ((), ())),
            preferred_element_type=jnp.float32,
        )
        diff = jnp.maximum((g2 + sq_c) + sq_r, EPS)
        prob = jnp.exp2((diff * jnp.float32(-1.4426950408889634))
                        * jax.lax.rsqrt(diff))
        isd = jnp.where(i == j, 1.0, 0.0)
        prob = jnp.maximum(prob, eye_ref[...] * isd)
        prob_ref[...] = prob
        pb_s[j, pl.ds(i * BM, BM), :] = prob.astype(jnp.bfloat16)

        ps = _rowfold(prob)

        @pl.when(j == 0)
        def _():
            rs_s[pl.ds(i * BM, BM), :] = ps

        @pl.when(j != 0)
        def _():
            rs_s[pl.ds(i * BM, BM), :] += ps

        @pl.when(j == NJ - 1)
        def _():
            deg = jnp.sum(rs_s[pl.ds(i * BM, BM), :], axis=1,
                          keepdims=True) - 1.0
            dinv = jnp.where(deg > 0.0, jax.lax.rsqrt(deg), 0.0)
            dinv_s[pl.ds(i * BM, BM), :] = dinv
            uh_s[pl.ds(i * BM, BM), :] = (
                dinv * x_ref[pl.ds(i * BM, BM), :]).astype(jnp.bfloat16)

    @pl.when(jnp.logical_and(t >= NI * NJ, t < NI * NJ + NI))
    def _():
        i = t - NI * NJ
        z = jnp.zeros((BM, 128), jnp.float32)
        for jj in range(NJ):
            z += jax.lax.dot_general(
                pb_s[jj, pl.ds(i * BM, BM), :],
                uh_s[pl.ds(jj * BN, BN), :],
                (((1,), (0,)), ((), ())),
                preferred_element_type=jnp.float32,
            )
        u_i = uh_s[pl.ds(i * BM, BM), :].astype(jnp.float32)
        dinv_i = dinv_s[pl.ds(i * BM, BM), :]
        tx1 = -(dinv_i * (z - u_i))
        tx1_s[pl.ds(i * BM, BM), :] = tx1
        u2h_s[pl.ds(i * BM, BM), :] = (dinv_i * tx1).astype(jnp.bfloat16)

    @pl.when(t >= NI * NJ + NI)
    def _():
        i = t - (NI * NJ + NI)
        z = jnp.zeros((BM, 128), jnp.float32)
        for jj in range(NJ):
            z += jax.lax.dot_general(
                pb_s[jj, pl.ds(i * BM, BM), :],
                u2h_s[pl.ds(jj * BN, BN), :],
                (((1,), (0,)), ((), ())),
                preferred_element_type=jnp.float32,
            )
        u2_i = u2h_s[pl.ds(i * BM, BM), :].astype(jnp.float32)
        dinv_i = dinv_s[pl.ds(i * BM, BM), :]
        lt = -(dinv_i * (z - u2_i))
        xi = x_ref[pl.ds(i * BM, BM), :]
        tx1i = tx1_s[pl.ds(i * BM, BM), :]
        tx2 = 2.0 * lt - xi
        o = jnp.dot(xi, w_ref[0], preferred_element_type=jnp.float32)
        o += jnp.dot(tx1i, w_ref[1], preferred_element_type=jnp.float32)
        o += jnp.dot(tx2, w_ref[2], preferred_element_type=jnp.float32)
        out_ref[...] = jnp.maximum(o + b_ref[...], 0.0)


def kernel(input, adj, lin_w, lin_b, cheb_w, cheb_b):
    x = input
    n, d = x.shape

    glh, sq, sqe = pl.pallas_call(
        _linear_kernel,
        grid=(n // BL,),
        in_specs=[
            pl.BlockSpec((BL, d), lambda i: (i, 0)),
            pl.BlockSpec((d, d), lambda i: (0, 0)),
            pl.BlockSpec((1, d), lambda i: (0, 0)),
        ],
        out_specs=[
            pl.BlockSpec((BL, d), lambda i: (i, 0)),
            pl.BlockSpec((BL, 1), lambda i: (i, 0)),
            pl.BlockSpec((BL, 1), lambda i: (i, 0)),
        ],
        out_shape=[
            jax.ShapeDtypeStruct((n, d), jnp.bfloat16),
            jax.ShapeDtypeStruct((n, 1), jnp.float32),
            jax.ShapeDtypeStruct((n, 1), jnp.float32),
        ],
        compiler_params=pltpu.CompilerParams(
            dimension_semantics=("parallel",)),
    )(x, lin_w, lin_b.reshape(1, d))

    NI, NJ = n // BM, n // BN
    sqt = sq.reshape(NJ, 1, BN)
    eye = jnp.eye(BM, dtype=jnp.float32)
    nt = NI * NJ + 2 * NI

    full = lambda t: (0, 0)

    prob, out = pl.pallas_call(
        functools.partial(_mega_kernel, NI=NI, NJ=NJ),
        grid=(nt,),
        in_specs=[
            pl.BlockSpec((n, d), full),
            pl.BlockSpec((n, 1), full),
            pl.BlockSpec((NJ, 1, BN), lambda t: (0, 0, 0)),

            pl.BlockSpec((BM, BN), full),
            pl.BlockSpec((n, d), full),
            pl.BlockSpec((3, d, d), lambda t: (0, 0, 0)),
            pl.BlockSpec((1, d), full),
        ],
        out_specs=[
            pl.BlockSpec(
                (BM, BN),
                lambda t, NI=NI, NJ=NJ: (
                    jnp.where(t < NI * NJ, t // NJ, NI - 1),
                    jnp.where(t < NI * NJ, t % NJ, NJ - 1)),
            ),
            pl.BlockSpec(
                (BM, d),
                lambda t, NI=NI, NJ=NJ: (
                    jnp.where(t >= NI * NJ + NI, t - (NI * NJ + NI), 0), 0),
            ),
        ],
        out_shape=[
            jax.ShapeDtypeStruct((n, n), jnp.float32),
            jax.ShapeDtypeStruct((n, d), jnp.float32),
        ],
        scratch_shapes=[
            pltpu.VMEM((NJ, n, BN), jnp.bfloat16),
            pltpu.VMEM((n, 128), jnp.float32),
            pltpu.VMEM((n, 1), jnp.float32),
            pltpu.VMEM((n, d), jnp.bfloat16),
            pltpu.VMEM((n, d), jnp.float32),
            pltpu.VMEM((n, d), jnp.bfloat16),
        ],
        compiler_params=pltpu.CompilerParams(
            dimension_semantics=("arbitrary",)),
    )(glh, sqe, sqt, eye, x, cheb_w, cheb_b.reshape(1, d))

    return out, prob


# speedup vs baseline: 1.1728x; 1.1281x over previous
import functools

import jax
import jax.numpy as jnp
from jax.experimental import pallas as pl
from jax.experimental.pallas import tpu as pltpu

EPS = 1.1920929e-07

BL = 512
BM = 512
BN = 1024


def _linear_kernel(x_ref, w_ref, b_ref, glh_ref, sq_ref, sqe_ref):
    gl = jnp.maximum(
        jnp.dot(x_ref[...], w_ref[...], preferred_element_type=jnp.float32)
        + b_ref[...],
        0.0,
    )
    glh_ref[...] = gl.astype(jnp.bfloat16)
    sq = jnp.sum(gl * gl, axis=1, keepdims=True)
    sq_ref[...] = sq
    sqe_ref[...] = sq + EPS


def _rowfold(p):
    acc = p[:, 0:128]
    for c in range(128, BN, 128):
        acc = acc + p[:, c:c + 128]
    return acc


def _mega_kernel(glh_ref, sqe_ref, sqt_ref, eye_ref, x_ref, w_ref, b_ref,
                 prob_ref, out_ref,
                 pb_s, rs_s, dinv_s, uh_s, tx1_s, u2h_s, *, NI, NJ):
    t = pl.program_id(0)

    @pl.when(t < NI * NJ)
    def _():
        i = t // NJ
        j = t - i * NJ
        glr = glh_ref[pl.ds(i * BM, BM), :]
        glc = glh_ref[pl.ds(j * BN, BN), :]
        sq_r = sqe_ref[pl.ds(i * BM, BM), :]
        sq_c = sqt_ref[j]
        g2 = jax.lax.dot_general(
            glr * jnp.bfloat16(-2.0), glc, (((1,), (1,)), ((), ())),
            preferred_element_type=jnp.float32,
        )
        diff = jnp.maximum((g2 + sq_c) + sq_r, EPS)
        prob = jnp.exp2((diff * jnp.float32(-1.4426950408889634))
                        * jax.lax.rsqrt(diff))
        isd = jnp.where(i // (BN // BM) == j, 1.0, 0.0)
        prob = jnp.maximum(prob, eye_ref[i % (BN // BM)] * isd)
        prob_ref[...] = prob
        pb_s[j, pl.ds(i * BM, BM), :] = prob.astype(jnp.bfloat16)

        ps = _rowfold(prob)

        @pl.when(j == 0)
        def _():
            rs_s[pl.ds(i * BM, BM), :] = ps

        @pl.when(j != 0)
        def _():
            rs_s[pl.ds(i * BM, BM), :] += ps

        @pl.when(j == NJ - 1)
        def _():
            deg = jnp.sum(rs_s[pl.ds(i * BM, BM), :], axis=1,
                          keepdims=True) - 1.0
            dinv = jnp.where(deg > 0.0, jax.lax.rsqrt(deg), 0.0)
            dinv_s[pl.ds(i * BM, BM), :] = dinv
            uh_s[pl.ds(i * BM, BM), :] = (
                dinv * x_ref[pl.ds(i * BM, BM), :]).astype(jnp.bfloat16)

    @pl.when(jnp.logical_and(t >= NI * NJ, t < NI * NJ + NI))
    def _():
        i = t - NI * NJ
        z = jnp.zeros((BM, 128), jnp.float32)
        for jj in range(NJ):
            z += jax.lax.dot_general(
                pb_s[jj, pl.ds(i * BM, BM), :],
                uh_s[pl.ds(jj * BN, BN), :],
                (((1,), (0,)), ((), ())),
                preferred_element_type=jnp.float32,
            )
        u_i = uh_s[pl.ds(i * BM, BM), :].astype(jnp.float32)
        dinv_i = dinv_s[pl.ds(i * BM, BM), :]
        tx1 = -(dinv_i * (z - u_i))
        tx1_s[pl.ds(i * BM, BM), :] = tx1
        u2h_s[pl.ds(i * BM, BM), :] = (dinv_i * tx1).astype(jnp.bfloat16)

    @pl.when(t >= NI * NJ + NI)
    def _():
        i = t - (NI * NJ + NI)
        z = jnp.zeros((BM, 128), jnp.float32)
        for jj in range(NJ):
            z += jax.lax.dot_general(
                pb_s[jj, pl.ds(i * BM, BM), :],
                u2h_s[pl.ds(jj * BN, BN), :],
                (((1,), (0,)), ((), ())),
                preferred_element_type=jnp.float32,
            )
        u2_i = u2h_s[pl.ds(i * BM, BM), :].astype(jnp.float32)
        dinv_i = dinv_s[pl.ds(i * BM, BM), :]
        lt = -(dinv_i * (z - u2_i))
        xi = x_ref[pl.ds(i * BM, BM), :]
        tx1i = tx1_s[pl.ds(i * BM, BM), :]
        tx2 = 2.0 * lt - xi
        o = jnp.dot(xi, w_ref[0], preferred_element_type=jnp.float32)
        o += jnp.dot(tx1i, w_ref[1], preferred_element_type=jnp.float32)
        o += jnp.dot(tx2, w_ref[2], preferred_element_type=jnp.float32)
        out_ref[...] = jnp.maximum(o + b_ref[...], 0.0)


def kernel(input, adj, lin_w, lin_b, cheb_w, cheb_b):
    x = input
    n, d = x.shape

    glh, sq, sqe = pl.pallas_call(
        _linear_kernel,
        grid=(n // BL,),
        in_specs=[
            pl.BlockSpec((BL, d), lambda i: (i, 0)),
            pl.BlockSpec((d, d), lambda i: (0, 0)),
            pl.BlockSpec((1, d), lambda i: (0, 0)),
        ],
        out_specs=[
            pl.BlockSpec((BL, d), lambda i: (i, 0)),
            pl.BlockSpec((BL, 1), lambda i: (i, 0)),
            pl.BlockSpec((BL, 1), lambda i: (i, 0)),
        ],
        out_shape=[
            jax.ShapeDtypeStruct((n, d), jnp.bfloat16),
            jax.ShapeDtypeStruct((n, 1), jnp.float32),
            jax.ShapeDtypeStruct((n, 1), jnp.float32),
        ],
        compiler_params=pltpu.CompilerParams(
            dimension_semantics=("parallel",)),
    )(x, lin_w, lin_b.reshape(1, d))

    NI, NJ = n // BM, n // BN
    sqt = sq.reshape(NJ, 1, BN)
    nslot = BN // BM
    eye = jnp.concatenate(
        [jnp.eye(BM, dtype=jnp.float32)] * nslot, axis=1)
    eye = jnp.stack([jnp.where(
        (jax.lax.broadcasted_iota(jnp.int32, (BM, BN), 1) // BM) == k,
        eye, 0.0) for k in range(nslot)])
    nt = NI * NJ + 2 * NI

    full = lambda t: (0, 0)

    prob, out = pl.pallas_call(
        functools.partial(_mega_kernel, NI=NI, NJ=NJ),
        grid=(nt,),
        in_specs=[
            pl.BlockSpec((n, d), full),
            pl.BlockSpec((n, 1), full),
            pl.BlockSpec((NJ, 1, BN), lambda t: (0, 0, 0)),

            pl.BlockSpec((nslot, BM, BN), lambda t: (0, 0, 0)),
            pl.BlockSpec((n, d), full),
            pl.BlockSpec((3, d, d), lambda t: (0, 0, 0)),
            pl.BlockSpec((1, d), full),
        ],
        out_specs=[
            pl.BlockSpec(
                (BM, BN),
                lambda t, NI=NI, NJ=NJ: (
                    jnp.where(t < NI * NJ, t // NJ, NI - 1),
                    jnp.where(t < NI * NJ, t % NJ, NJ - 1)),
            ),
            pl.BlockSpec(
                (BM, d),
                lambda t, NI=NI, NJ=NJ: (
                    jnp.where(t >= NI * NJ + NI, t - (NI * NJ + NI), 0), 0),
            ),
        ],
        out_shape=[
            jax.ShapeDtypeStruct((n, n), jnp.float32),
            jax.ShapeDtypeStruct((n, d), jnp.float32),
        ],
        scratch_shapes=[
            pltpu.VMEM((NJ, n, BN), jnp.bfloat16),
            pltpu.VMEM((n, 128), jnp.float32),
            pltpu.VMEM((n, 1), jnp.float32),
            pltpu.VMEM((n, d), jnp.bfloat16),
            pltpu.VMEM((n, d), jnp.float32),
            pltpu.VMEM((n, d), jnp.bfloat16),
        ],
        compiler_params=pltpu.CompilerParams(
            dimension_semantics=("arbitrary",)),
    )(glh, sqe, sqt, eye, x, cheb_w, cheb_b.reshape(1, d))

    return out, prob
